# baseline (speedup 1.0000x reference)
import functools

import jax
import jax.numpy as jnp
from jax import lax
from jax.experimental import pallas as pl
from jax.experimental.pallas import tpu as pltpu
from jax.experimental.pallas import tpu_sc as plsc

N_NODES = 10000
N_EDGES = 320000
D = 128

NUM_CORES = 2
NUM_SUBCORES = 16
NUM_TILES = NUM_CORES * NUM_SUBCORES

CHUNK = 128
CHUNKS_PER_TILE = 80
EDGES_PER_TILE = CHUNK * CHUNKS_PER_TILE
N_PAD_EDGES = EDGES_PER_TILE * NUM_TILES
NB = 1
TRASH_ROW = N_NODES
N_ACC = 10240
ROWS_PER_TILE_ZERO = N_ACC // NUM_SUBCORES
ROWS_PER_TILE_OUT = 624

_MM_BLOCK = 1000


def _stage1_body(x_ref, wlT_ref, bl_ref, wrT_ref, xl_ref, outr_ref):
    x = x_ref[...]
    xl_ref[...] = (
        jnp.dot(x, wlT_ref[...], preferred_element_type=jnp.float32)
        + bl_ref[...]
    )
    outr_ref[...] = jnp.dot(x, wrT_ref[...], preferred_element_type=jnp.float32)


def _stage1(x, wlT, bl2d, wrT):
    grid = (N_NODES // _MM_BLOCK,)
    return pl.pallas_call(
        _stage1_body,
        grid=grid,
        in_specs=[
            pl.BlockSpec((_MM_BLOCK, D), lambda i: (i, 0)),
            pl.BlockSpec((D, D), lambda i: (0, 0)),
            pl.BlockSpec((1, D), lambda i: (0, 0)),
            pl.BlockSpec((D, D), lambda i: (0, 0)),
        ],
        out_specs=[
            pl.BlockSpec((_MM_BLOCK, D), lambda i: (i, 0)),
            pl.BlockSpec((_MM_BLOCK, D), lambda i: (i, 0)),
        ],
        out_shape=[
            jax.ShapeDtypeStruct((N_NODES, D), jnp.float32),
            jax.ShapeDtypeStruct((N_NODES, D), jnp.float32),
        ],
    )(x, wlT, bl2d, wrT)


def _sc_body(
    xl_hbm, src_hbm, dst_hbm, out_hbm,
    isrc, idst, big,
    acc_sh,
    gsem,
):
    cid = lax.axis_index("c")
    sid = lax.axis_index("s")
    tile = cid * NUM_SUBCORES + sid

    pltpu.sync_copy(src_hbm.at[pl.ds(tile * CHUNKS_PER_TILE, CHUNKS_PER_TILE)], isrc)
    pltpu.sync_copy(dst_hbm.at[pl.ds(tile * CHUNKS_PER_TILE, CHUNKS_PER_TILE)], idst)

    zeros16 = jnp.zeros((16,), jnp.float32)

    def zero_body(i, carry):
        r = i // (D // 16)
        c = i % (D // 16)
        big[r, pl.ds(c * 16, 16)] = zeros16
        return carry

    lax.fori_loop(0, CHUNK * (D // 16), zero_body, 0)

    def zcopy_body(j, carry):
        pltpu.sync_copy(
            big.at[pl.ds(0, CHUNK)],
            acc_sh.at[pl.ds(sid * ROWS_PER_TILE_ZERO + j * CHUNK, CHUNK)],
        )
        return carry

    lax.fori_loop(0, ROWS_PER_TILE_ZERO // CHUNK, zcopy_body, 0)
    plsc.subcore_barrier()

    def body(i, carry):
        def gfire(b, carry2):
            pltpu.async_copy(
                xl_hbm.at[isrc.at[i * NB + b]],
                big.at[pl.ds(b * CHUNK, CHUNK)],
                gsem,
            )
            return carry2

        lax.fori_loop(0, NB, gfire, 0)

        def gdrain(b, carry2):
            pltpu.make_async_copy(
                xl_hbm.at[isrc.at[i * NB + b]],
                big.at[pl.ds(b * CHUNK, CHUNK)],
                gsem,
            ).wait()
            return carry2

        lax.fori_loop(0, NB, gdrain, 0)

        def sfire(b, carry2):
            pltpu.sync_copy(
                big.at[pl.ds(b * CHUNK, CHUNK)],
                acc_sh.at[idst.at[i * NB + b]],
                add=True,
            )
            return carry2

        lax.fori_loop(0, NB, sfire, 0)
        return carry

    lax.fori_loop(0, CHUNKS_PER_TILE // NB, body, 0)
    plsc.subcore_barrier()

    row0 = sid * ROWS_PER_TILE_OUT
    pltpu.sync_copy(
        acc_sh.at[pl.ds(row0, ROWS_PER_TILE_OUT)],
        out_hbm.at[pl.ds(cid * N_NODES + row0, ROWS_PER_TILE_OUT)],
    )
    tail0 = NUM_SUBCORES * ROWS_PER_TILE_OUT
    tail = N_NODES - tail0

    @pl.when(sid == NUM_SUBCORES - 1)
    def _write_tail():
        pltpu.sync_copy(
            acc_sh.at[pl.ds(tail0, tail)],
            out_hbm.at[pl.ds(cid * N_NODES + tail0, tail)],
        )


_sc_stage = functools.partial(
    pl.kernel,
    out_type=jax.ShapeDtypeStruct((NUM_CORES * N_NODES, D), jnp.float32),
    mesh=plsc.VectorSubcoreMesh(core_axis_name="c", subcore_axis_name="s"),
    scratch_types=(
        [
            pltpu.VMEM((CHUNKS_PER_TILE, CHUNK), jnp.int32),
            pltpu.VMEM((CHUNKS_PER_TILE, CHUNK), jnp.int32),
        ]
        + [pltpu.VMEM((NB * CHUNK, D), jnp.float32)]
        + [pltpu.VMEM_SHARED((N_ACC, D), jnp.float32)]
        + [pltpu.SemaphoreType.DMA]
    ),
)(_sc_body)


def _stage3_body(p0_ref, p1_ref, outr_ref, out_ref):
    out_ref[...] = p0_ref[...] + p1_ref[...] + outr_ref[...]


def _stage3(p0, p1, outr):
    grid = (N_NODES // _MM_BLOCK,)
    spec = pl.BlockSpec((_MM_BLOCK, D), lambda i: (i, 0))
    return pl.pallas_call(
        _stage3_body,
        grid=grid,
        in_specs=[spec, spec, spec],
        out_specs=spec,
        out_shape=jax.ShapeDtypeStruct((N_NODES, D), jnp.float32),
    )(p0, p1, outr)


def kernel(x, edge_index, W_l, b_l, W_r):
    src = edge_index[0]
    dst = edge_index[1]
    pad = N_PAD_EDGES - N_EDGES
    src_pad = jnp.concatenate([src, jnp.zeros((pad,), jnp.int32)]).reshape(-1, CHUNK)
    dst_pad = jnp.concatenate([dst, jnp.full((pad,), TRASH_ROW, jnp.int32)]).reshape(-1, CHUNK)

    xl, outr = _stage1(x, W_l.T, b_l.reshape(1, D), W_r.T)
    parts = _sc_stage(xl, src_pad, dst_pad)
    return _stage3(parts[:N_NODES], parts[N_NODES:], outr)


# speedup vs baseline: 1.0000x; 1.0000x over previous
import functools

import jax
import jax.numpy as jnp
from jax import lax
from jax.experimental import pallas as pl
from jax.experimental.pallas import tpu as pltpu
from jax.experimental.pallas import tpu_sc as plsc

N_NODES = 10000
N_EDGES = 320000
D = 128

NUM_CORES = 2
NUM_SUBCORES = 16
NUM_TILES = NUM_CORES * NUM_SUBCORES

CHUNK = 128
CHUNKS_PER_TILE = 80
EDGES_PER_TILE = CHUNK * CHUNKS_PER_TILE
N_PAD_EDGES = EDGES_PER_TILE * NUM_TILES
NB = 1
TRASH_ROW = N_NODES
N_ACC = 10240
ROWS_PER_TILE_ZERO = N_ACC // NUM_SUBCORES
ROWS_PER_TILE_OUT = 624

_MM_BLOCK = 1000


def _stage1_body(x_ref, wlT_ref, bl_ref, wrT_ref, xl_ref, outr_ref):
    x = x_ref[...]
    xl_ref[...] = (
        jnp.dot(x, wlT_ref[...], preferred_element_type=jnp.float32)
        + bl_ref[...]
    )
    outr_ref[...] = jnp.dot(x, wrT_ref[...], preferred_element_type=jnp.float32)


def _stage1(x, wlT, bl2d, wrT):
    grid = (N_NODES // _MM_BLOCK,)
    return pl.pallas_call(
        _stage1_body,
        grid=grid,
        in_specs=[
            pl.BlockSpec((_MM_BLOCK, D), lambda i: (i, 0)),
            pl.BlockSpec((D, D), lambda i: (0, 0)),
            pl.BlockSpec((1, D), lambda i: (0, 0)),
            pl.BlockSpec((D, D), lambda i: (0, 0)),
        ],
        out_specs=[
            pl.BlockSpec((_MM_BLOCK, D), lambda i: (i, 0)),
            pl.BlockSpec((_MM_BLOCK, D), lambda i: (i, 0)),
        ],
        out_shape=[
            jax.ShapeDtypeStruct((N_NODES, D), jnp.float32),
            jax.ShapeDtypeStruct((N_NODES, D), jnp.float32),
        ],
    )(x, wlT, bl2d, wrT)


def _sc_body(
    xl_hbm, src_hbm, dst_hbm, out_hbm,
    isrc, idst, big,
    acc_sh,
    gsem,
):
    cid = lax.axis_index("c")
    sid = lax.axis_index("s")
    tile = cid * NUM_SUBCORES + sid

    pltpu.sync_copy(src_hbm.at[pl.ds(tile * CHUNKS_PER_TILE, CHUNKS_PER_TILE)], isrc)
    pltpu.sync_copy(dst_hbm.at[pl.ds(tile * CHUNKS_PER_TILE, CHUNKS_PER_TILE)], idst)

    zeros16 = jnp.zeros((16,), jnp.float32)

    def zero_body(i, carry):
        r = i // (D // 16)
        c = i % (D // 16)
        big[r, pl.ds(c * 16, 16)] = zeros16
        return carry

    lax.fori_loop(0, CHUNK * (D // 16), zero_body, 0)

    def zcopy_body(j, carry):
        pltpu.sync_copy(
            big.at[pl.ds(0, CHUNK)],
            acc_sh.at[pl.ds(sid * ROWS_PER_TILE_ZERO + j * CHUNK, CHUNK)],
        )
        return carry

    lax.fori_loop(0, ROWS_PER_TILE_ZERO // CHUNK, zcopy_body, 0)
    plsc.subcore_barrier()

    def body(i, carry):
        pltpu.async_copy(xl_hbm.at[isrc.at[i]], big, gsem).wait()
        pltpu.sync_copy(big, acc_sh.at[idst.at[i]], add=True)
        return carry

    lax.fori_loop(0, CHUNKS_PER_TILE, body, 0)
    plsc.subcore_barrier()

    row0 = sid * ROWS_PER_TILE_OUT
    pltpu.sync_copy(
        acc_sh.at[pl.ds(row0, ROWS_PER_TILE_OUT)],
        out_hbm.at[pl.ds(cid * N_NODES + row0, ROWS_PER_TILE_OUT)],
    )
    tail0 = NUM_SUBCORES * ROWS_PER_TILE_OUT
    tail = N_NODES - tail0

    @pl.when(sid == NUM_SUBCORES - 1)
    def _write_tail():
        pltpu.sync_copy(
            acc_sh.at[pl.ds(tail0, tail)],
            out_hbm.at[pl.ds(cid * N_NODES + tail0, tail)],
        )


_sc_stage = functools.partial(
    pl.kernel,
    out_type=jax.ShapeDtypeStruct((NUM_CORES * N_NODES, D), jnp.float32),
    mesh=plsc.VectorSubcoreMesh(core_axis_name="c", subcore_axis_name="s"),
    scratch_types=(
        [
            pltpu.VMEM((CHUNKS_PER_TILE, CHUNK), jnp.int32),
            pltpu.VMEM((CHUNKS_PER_TILE, CHUNK), jnp.int32),
        ]
        + [pltpu.VMEM((CHUNK, D), jnp.float32)]
        + [pltpu.VMEM_SHARED((N_ACC, D), jnp.float32)]
        + [pltpu.SemaphoreType.DMA]
    ),
)(_sc_body)


def _stage3_body(p0_ref, p1_ref, outr_ref, out_ref):
    out_ref[...] = p0_ref[...] + p1_ref[...] + outr_ref[...]


def _stage3(p0, p1, outr):
    grid = (N_NODES // _MM_BLOCK,)
    spec = pl.BlockSpec((_MM_BLOCK, D), lambda i: (i, 0))
    return pl.pallas_call(
        _stage3_body,
        grid=grid,
        in_specs=[spec, spec, spec],
        out_specs=spec,
        out_shape=jax.ShapeDtypeStruct((N_NODES, D), jnp.float32),
    )(p0, p1, outr)


def kernel(x, edge_index, W_l, b_l, W_r):
    src = edge_index[0]
    dst = edge_index[1]
    pad = N_PAD_EDGES - N_EDGES
    src_pad = jnp.concatenate([src, jnp.zeros((pad,), jnp.int32)]).reshape(-1, CHUNK)
    dst_pad = jnp.concatenate([dst, jnp.full((pad,), TRASH_ROW, jnp.int32)]).reshape(-1, CHUNK)

    xl, outr = _stage1(x, W_l.T, b_l.reshape(1, D), W_r.T)
    parts = _sc_stage(xl, src_pad, dst_pad)
    return _stage3(parts[:N_NODES], parts[N_NODES:], outr)


# speedup vs baseline: 1.0871x; 1.0871x over previous
import functools

import jax
import jax.numpy as jnp
from jax import lax
from jax.experimental import pallas as pl
from jax.experimental.pallas import tpu as pltpu
from jax.experimental.pallas import tpu_sc as plsc

N_NODES = 10000
N_EDGES = 320000
D = 128

NUM_CORES = 2
NUM_SUBCORES = 16
NUM_TILES = NUM_CORES * NUM_SUBCORES

CHUNK = 128
CHUNKS_PER_TILE = 80
EDGES_PER_TILE = CHUNK * CHUNKS_PER_TILE
N_PAD_EDGES = EDGES_PER_TILE * NUM_TILES
TRASH_ROW = N_NODES
N_ACC = 10240
ROWS_PER_TILE_ZERO = N_ACC // NUM_SUBCORES
ROWS_PER_TILE_OUT = 624

_MM_BLOCK = 1000


def _stage1_body(x_ref, wlT_ref, bl_ref, wrT_ref, xl_ref, outr_ref):
    x = x_ref[...]
    xl_ref[...] = (
        jnp.dot(x, wlT_ref[...], preferred_element_type=jnp.float32)
        + bl_ref[...]
    )
    outr_ref[...] = jnp.dot(x, wrT_ref[...], preferred_element_type=jnp.float32)


def _stage1(x, wlT, bl2d, wrT):
    grid = (N_NODES // _MM_BLOCK,)
    return pl.pallas_call(
        _stage1_body,
        grid=grid,
        in_specs=[
            pl.BlockSpec((_MM_BLOCK, D), lambda i: (i, 0)),
            pl.BlockSpec((D, D), lambda i: (0, 0)),
            pl.BlockSpec((1, D), lambda i: (0, 0)),
            pl.BlockSpec((D, D), lambda i: (0, 0)),
        ],
        out_specs=[
            pl.BlockSpec((_MM_BLOCK, D), lambda i: (i, 0)),
            pl.BlockSpec((_MM_BLOCK, D), lambda i: (i, 0)),
        ],
        out_shape=[
            jax.ShapeDtypeStruct((N_NODES, D), jnp.float32),
            jax.ShapeDtypeStruct((N_NODES, D), jnp.float32),
        ],
    )(x, wlT, bl2d, wrT)


def _sc_body(
    xl_hbm, src_hbm, dst_hbm, out_hbm,
    srcA, dstA, srcB, dstB, rA, rB,
    acc_sh,
    gA, gB,
):
    cid = lax.axis_index("c")
    sid = lax.axis_index("s")
    tile = cid * NUM_SUBCORES + sid
    e0 = tile * EDGES_PER_TILE

    zeros16 = jnp.zeros((16,), jnp.float32)

    def zero_body(i, carry):
        r = i // (D // 16)
        c = i % (D // 16)
        rA[r, pl.ds(c * 16, 16)] = zeros16
        return carry

    lax.fori_loop(0, CHUNK * (D // 16), zero_body, 0)

    def zcopy_body(j, carry):
        pltpu.sync_copy(
            rA,
            acc_sh.at[pl.ds(sid * ROWS_PER_TILE_ZERO + j * CHUNK, CHUNK)],
        )
        return carry

    lax.fori_loop(0, ROWS_PER_TILE_ZERO // CHUNK, zcopy_body, 0)
    plsc.subcore_barrier()

    pltpu.sync_copy(src_hbm.at[pl.ds(e0, CHUNK)], srcA)
    pltpu.sync_copy(dst_hbm.at[pl.ds(e0, CHUNK)], dstA)
    pltpu.async_copy(xl_hbm.at[srcA], rA, gA)

    def body(i, carry):
        c1 = 2 * i + 1
        c2 = 2 * i + 2
        pltpu.sync_copy(src_hbm.at[pl.ds(e0 + c1 * CHUNK, CHUNK)], srcB)
        pltpu.sync_copy(dst_hbm.at[pl.ds(e0 + c1 * CHUNK, CHUNK)], dstB)
        pltpu.make_async_copy(xl_hbm.at[srcA], rA, gA).wait()
        pltpu.async_copy(xl_hbm.at[srcB], rB, gB)
        pltpu.sync_copy(rA, acc_sh.at[dstA], add=True)
        pltpu.sync_copy(src_hbm.at[pl.ds(e0 + c2 * CHUNK, CHUNK)], srcA)
        pltpu.sync_copy(dst_hbm.at[pl.ds(e0 + c2 * CHUNK, CHUNK)], dstA)
        pltpu.make_async_copy(xl_hbm.at[srcB], rB, gB).wait()
        pltpu.async_copy(xl_hbm.at[srcA], rA, gA)
        pltpu.sync_copy(rB, acc_sh.at[dstB], add=True)
        return carry

    lax.fori_loop(0, CHUNKS_PER_TILE // 2, body, 0)
    pltpu.make_async_copy(xl_hbm.at[srcA], rA, gA).wait()
    plsc.subcore_barrier()

    row0 = sid * ROWS_PER_TILE_OUT
    pltpu.sync_copy(
        acc_sh.at[pl.ds(row0, ROWS_PER_TILE_OUT)],
        out_hbm.at[pl.ds(cid * N_NODES + row0, ROWS_PER_TILE_OUT)],
    )
    tail0 = NUM_SUBCORES * ROWS_PER_TILE_OUT
    tail = N_NODES - tail0

    @pl.when(sid == NUM_SUBCORES - 1)
    def _write_tail():
        pltpu.sync_copy(
            acc_sh.at[pl.ds(tail0, tail)],
            out_hbm.at[pl.ds(cid * N_NODES + tail0, tail)],
        )


_sc_stage = functools.partial(
    pl.kernel,
    out_type=jax.ShapeDtypeStruct((NUM_CORES * N_NODES, D), jnp.float32),
    mesh=plsc.VectorSubcoreMesh(core_axis_name="c", subcore_axis_name="s"),
    scratch_types=(
        [pltpu.VMEM((CHUNK,), jnp.int32) for _ in range(4)]
        + [pltpu.VMEM((CHUNK, D), jnp.float32) for _ in range(2)]
        + [pltpu.VMEM_SHARED((N_ACC, D), jnp.float32)]
        + [pltpu.SemaphoreType.DMA for _ in range(2)]
    ),
)(_sc_body)


def _stage3_body(p0_ref, p1_ref, outr_ref, out_ref):
    out_ref[...] = p0_ref[...] + p1_ref[...] + outr_ref[...]


def _stage3(p0, p1, outr):
    grid = (N_NODES // _MM_BLOCK,)
    spec = pl.BlockSpec((_MM_BLOCK, D), lambda i: (i, 0))
    return pl.pallas_call(
        _stage3_body,
        grid=grid,
        in_specs=[spec, spec, spec],
        out_specs=spec,
        out_shape=jax.ShapeDtypeStruct((N_NODES, D), jnp.float32),
    )(p0, p1, outr)


def kernel(x, edge_index, W_l, b_l, W_r):
    src = edge_index[0]
    dst = edge_index[1]
    pad = N_PAD_EDGES + CHUNK - N_EDGES
    src_pad = jnp.concatenate([src, jnp.zeros((pad,), jnp.int32)])
    dst_pad = jnp.concatenate([dst, jnp.full((pad,), TRASH_ROW, jnp.int32)])

    xl, outr = _stage1(x, W_l.T, b_l.reshape(1, D), W_r.T)
    parts = _sc_stage(xl, src_pad, dst_pad)
    return _stage3(parts[:N_NODES], parts[N_NODES:], outr)
